# baseline (speedup 1.0000x reference)
import functools

import jax
import jax.numpy as jnp
from jax import lax
from jax.experimental import pallas as pl
from jax.experimental.pallas import tpu as pltpu
from jax.experimental.pallas import tpu_sc as plsc


_CHUNK = 16


def _sc_gather(table, flat_idx, offset, nrows):
  d = table.shape[1]
  mesh = plsc.VectorSubcoreMesh(core_axis_name="core", subcore_axis_name="subcore")
  nw = mesh.num_cores * mesh.num_subcores
  per_w = nrows // nw
  nchunk = per_w // _CHUNK

  @functools.partial(
      pl.kernel,
      out_type=jax.ShapeDtypeStruct((nrows, d), table.dtype),
      mesh=mesh,
      scratch_types=[
          pltpu.VMEM((per_w,), jnp.int32),
          pltpu.VMEM((_CHUNK, d), table.dtype),
          pltpu.VMEM((_CHUNK, d), table.dtype),
          pltpu.SemaphoreType.DMA,
          pltpu.SemaphoreType.DMA,
          pltpu.SemaphoreType.DMA,
          pltpu.SemaphoreType.DMA,
      ],
  )
  def gather_kernel(x_hbm, i_hbm, o_hbm, idx_v, rows0, rows1, g0, g1, s0, s1):
    wid = lax.axis_index("subcore") * mesh.num_cores + lax.axis_index("core")
    base = wid * per_w
    pltpu.sync_copy(i_hbm.at[pl.ds(offset + base, per_w)], idx_v)

    bufs = (rows0, rows1)
    gsem = (g0, g1)
    ssem = (s0, s1)

    def start_gather(j, b):
      return pltpu.async_copy(
          x_hbm.at[idx_v.at[pl.ds(j * _CHUNK, _CHUNK)]], bufs[b], gsem[b]
      )

    def start_store(j, b):
      return pltpu.async_copy(
          bufs[b], o_hbm.at[pl.ds(base + j * _CHUNK, _CHUNK)], ssem[b]
      )

    g_h = [start_gather(0, 0), None]
    s_h = [None, None]
    for j in range(nchunk):
      b = j % 2
      if j + 1 < nchunk:
        if s_h[1 - b] is not None:
          s_h[1 - b].wait()
        g_h[1 - b] = start_gather(j + 1, 1 - b)
      g_h[b].wait()
      s_h[b] = start_store(j, b)
    for h in s_h:
      if h is not None:
        h.wait()

  return gather_kernel(table, flat_idx)


_KC = 256


def _mm_body(y_in_ref, xg_ref, w_ref, o_ref, wb_ref):
  del y_in_ref

  @pl.when((pl.program_id(1) == 0) & (pl.program_id(2) == 0))
  def _cast_w():
    wb_ref[...] = w_ref[0].astype(jnp.bfloat16)

  a = xg_ref[0, 0].astype(jnp.bfloat16)
  o_ref[0, 0] = jnp.dot(a, wb_ref[...], preferred_element_type=jnp.float32)


def _tc_matmul_chunk(y_prev, xg, W, e0, epc, alias):
  _, B, K, I = xg.shape
  E, J = W.shape[0], W.shape[2]
  kwargs = {}
  if alias:
    kwargs["input_output_aliases"] = {0: 0}
  return pl.pallas_call(
      _mm_body,
      grid=(epc, B, K // _KC),
      in_specs=[
          pl.BlockSpec(memory_space=pl.ANY),
          pl.BlockSpec((1, 1, _KC, I), lambda e, b, k: (e, b, k, 0)),
          pl.BlockSpec((1, I, J), lambda e, b, k: (e0 + e, 0, 0)),
      ],
      out_specs=pl.BlockSpec((1, 1, _KC, J), lambda e, b, k: (b, e0 + e, k, 0)),
      out_shape=jax.ShapeDtypeStruct((B, E, K, J), jnp.float32),
      scratch_shapes=[pltpu.VMEM((I, J), jnp.bfloat16)],
      compiler_params=pltpu.CompilerParams(
          dimension_semantics=("arbitrary", "arbitrary", "arbitrary"),
      ),
      **kwargs,
  )(y_prev, xg, W)


_EPC = 2


def kernel(x, Ind, W):
  B, T, I = x.shape
  E, K = Ind.shape[1], Ind.shape[2]
  J = W.shape[2]
  table = x.reshape(B * T, I)
  flat_idx = (
      jnp.arange(B, dtype=jnp.int32)[None, :, None] * T
      + jnp.transpose(Ind, (1, 0, 2))
  ).reshape(E * B * K)

  nch = E // _EPC
  rows_per_chunk = _EPC * B * K
  xgs = [
      _sc_gather(table, flat_idx, c * rows_per_chunk, rows_per_chunk)
      for c in range(nch)
  ]
  del J
  y = None
  for c in range(nch):
    xg = xgs[c].reshape(_EPC, B, K, I)
    if y is None:
      dummy = jnp.zeros((8, 128), jnp.float32)
      y = _tc_matmul_chunk(dummy, xg, W, 0, _EPC, alias=False)
    else:
      y = _tc_matmul_chunk(y, xg, W, c * _EPC, _EPC, alias=True)
  return y


# speedup vs baseline: 1.0285x; 1.0285x over previous
import functools

import jax
import jax.numpy as jnp
from jax import lax
from jax.experimental import pallas as pl
from jax.experimental.pallas import tpu as pltpu
from jax.experimental.pallas import tpu_sc as plsc


_CHUNK = 16


def _sc_gather(table, flat_idx, offset, nrows):
  d = table.shape[1]
  mesh = plsc.VectorSubcoreMesh(core_axis_name="core", subcore_axis_name="subcore")
  nw = mesh.num_cores * mesh.num_subcores
  per_w = nrows // nw
  nchunk = per_w // _CHUNK

  @functools.partial(
      pl.kernel,
      out_type=jax.ShapeDtypeStruct((nrows, d), table.dtype),
      mesh=mesh,
      scratch_types=[
          pltpu.VMEM((per_w,), jnp.int32),
          pltpu.VMEM((_CHUNK, d), table.dtype),
          pltpu.VMEM((_CHUNK, d), table.dtype),
          pltpu.SemaphoreType.DMA,
          pltpu.SemaphoreType.DMA,
          pltpu.SemaphoreType.DMA,
          pltpu.SemaphoreType.DMA,
      ],
  )
  def gather_kernel(x_hbm, i_hbm, o_hbm, idx_v, rows0, rows1, g0, g1, s0, s1):
    wid = lax.axis_index("subcore") * mesh.num_cores + lax.axis_index("core")
    base = wid * per_w
    pltpu.sync_copy(i_hbm.at[pl.ds(offset + base, per_w)], idx_v)

    bufs = (rows0, rows1)
    gsem = (g0, g1)
    ssem = (s0, s1)

    def start_gather(j, b):
      return pltpu.async_copy(
          x_hbm.at[idx_v.at[pl.ds(j * _CHUNK, _CHUNK)]], bufs[b], gsem[b]
      )

    def start_store(j, b):
      return pltpu.async_copy(
          bufs[b], o_hbm.at[pl.ds(base + j * _CHUNK, _CHUNK)], ssem[b]
      )

    g_h = [start_gather(0, 0), None]
    s_h = [None, None]
    for j in range(nchunk):
      b = j % 2
      if j + 1 < nchunk:
        if s_h[1 - b] is not None:
          s_h[1 - b].wait()
        g_h[1 - b] = start_gather(j + 1, 1 - b)
      g_h[b].wait()
      s_h[b] = start_store(j, b)
    for h in s_h:
      if h is not None:
        h.wait()

  return gather_kernel(table, flat_idx)


def _mm_body(y_in_ref, xg_ref, wa_ref, wb_ref, o_ref):
  del y_in_ref
  a = xg_ref[0, 0].astype(jnp.bfloat16)
  jh = wa_ref.shape[2]
  o_ref[0, 0, :, :jh] = jnp.dot(
      a, wa_ref[0].astype(jnp.bfloat16), preferred_element_type=jnp.float32)
  o_ref[0, 0, :, jh:] = jnp.dot(
      a, wb_ref[0].astype(jnp.bfloat16), preferred_element_type=jnp.float32)


def _tc_matmul_chunk(y_prev, xg, W, e0, epc, alias):
  _, B, K, I = xg.shape
  E, J = W.shape[0], W.shape[2]
  kwargs = {}
  if alias:
    kwargs["input_output_aliases"] = {0: 0}
  return pl.pallas_call(
      _mm_body,
      grid=(epc, B),
      in_specs=[
          pl.BlockSpec(memory_space=pl.ANY),
          pl.BlockSpec((1, 1, K, I), lambda e, b: (e, b, 0, 0)),
          pl.BlockSpec((1, I, J // 2), lambda e, b: (e0 + e, 0, 0)),
          pl.BlockSpec((1, I, J // 2), lambda e, b: (e0 + e, 0, 1)),
      ],
      out_specs=pl.BlockSpec((1, 1, K, J), lambda e, b: (b, e0 + e, 0, 0)),
      out_shape=jax.ShapeDtypeStruct((B, E, K, J), jnp.float32),
      compiler_params=pltpu.CompilerParams(
          dimension_semantics=("arbitrary", "arbitrary"),
      ),
      **kwargs,
  )(y_prev, xg, W, W)


_EPC = 2


def kernel(x, Ind, W):
  B, T, I = x.shape
  E, K = Ind.shape[1], Ind.shape[2]
  J = W.shape[2]
  table = x.reshape(B * T, I)
  flat_idx = (
      jnp.arange(B, dtype=jnp.int32)[None, :, None] * T
      + jnp.transpose(Ind, (1, 0, 2))
  ).reshape(E * B * K)

  nch = E // _EPC
  rows_per_chunk = _EPC * B * K
  xgs = [
      _sc_gather(table, flat_idx, c * rows_per_chunk, rows_per_chunk)
      for c in range(nch)
  ]
  del J
  y = None
  for c in range(nch):
    xg = xgs[c].reshape(_EPC, B, K, I)
    if y is None:
      dummy = jnp.zeros((8, 128), jnp.float32)
      y = _tc_matmul_chunk(dummy, xg, W, 0, _EPC, alias=False)
    else:
      y = _tc_matmul_chunk(y, xg, W, c * _EPC, _EPC, alias=True)
  return y


# speedup vs baseline: 1.0786x; 1.0486x over previous
import functools

import jax
import jax.numpy as jnp
from jax import lax
from jax.experimental import pallas as pl
from jax.experimental.pallas import tpu as pltpu
from jax.experimental.pallas import tpu_sc as plsc


_CHUNK = 32


def _sc_gather(table, flat_idx, offset, nrows):
  d = table.shape[1]
  mesh = plsc.VectorSubcoreMesh(core_axis_name="core", subcore_axis_name="subcore")
  nw = mesh.num_cores * mesh.num_subcores
  per_w = nrows // nw
  nchunk = per_w // _CHUNK

  @functools.partial(
      pl.kernel,
      out_type=jax.ShapeDtypeStruct((nrows, d), table.dtype),
      mesh=mesh,
      scratch_types=[
          pltpu.VMEM((per_w,), jnp.int32),
          pltpu.VMEM((_CHUNK, d), table.dtype),
          pltpu.VMEM((_CHUNK, d), table.dtype),
          pltpu.SemaphoreType.DMA,
          pltpu.SemaphoreType.DMA,
          pltpu.SemaphoreType.DMA,
          pltpu.SemaphoreType.DMA,
      ],
  )
  def gather_kernel(x_hbm, i_hbm, o_hbm, idx_v, rows0, rows1, g0, g1, s0, s1):
    wid = lax.axis_index("subcore") * mesh.num_cores + lax.axis_index("core")
    base = wid * per_w
    pltpu.sync_copy(i_hbm.at[pl.ds(offset + base, per_w)], idx_v)

    bufs = (rows0, rows1)
    gsem = (g0, g1)
    ssem = (s0, s1)

    def start_gather(j, b):
      return pltpu.async_copy(
          x_hbm.at[idx_v.at[pl.ds(j * _CHUNK, _CHUNK)]], bufs[b], gsem[b]
      )

    def start_store(j, b):
      return pltpu.async_copy(
          bufs[b], o_hbm.at[pl.ds(base + j * _CHUNK, _CHUNK)], ssem[b]
      )

    g_h = [start_gather(0, 0), None]
    s_h = [None, None]
    for j in range(nchunk):
      b = j % 2
      if j + 1 < nchunk:
        if s_h[1 - b] is not None:
          s_h[1 - b].wait()
        g_h[1 - b] = start_gather(j + 1, 1 - b)
      g_h[b].wait()
      s_h[b] = start_store(j, b)
    for h in s_h:
      if h is not None:
        h.wait()

  return gather_kernel(table, flat_idx)


def _mm_body(y_in_ref, xg_ref, wlo_ref, whi_ref, o_ref):
  del y_in_ref
  w = xg_ref[0, 0]
  a_lo = lax.bitcast_convert_type(w << 16, jnp.float32).astype(jnp.bfloat16)
  a_hi = lax.bitcast_convert_type(
      w & jnp.int32(-65536), jnp.float32).astype(jnp.bfloat16)
  o_ref[0, 0] = (
      jnp.dot(a_lo, wlo_ref[0].astype(jnp.bfloat16),
              preferred_element_type=jnp.float32)
      + jnp.dot(a_hi, whi_ref[0].astype(jnp.bfloat16),
                preferred_element_type=jnp.float32)
  )


def _tc_matmul_chunk(y_prev, xg, W, e0, epc, alias):
  _, B, K, Ih = xg.shape
  E, I, J = W.shape
  kwargs = {}
  if alias:
    kwargs["input_output_aliases"] = {0: 0}
  return pl.pallas_call(
      _mm_body,
      grid=(epc, B),
      in_specs=[
          pl.BlockSpec(memory_space=pl.ANY),
          pl.BlockSpec((1, 1, K, Ih), lambda e, b: (e, b, 0, 0)),
          pl.BlockSpec((1, I // 2, J), lambda e, b: (e0 + e, 0, 0)),
          pl.BlockSpec((1, I // 2, J), lambda e, b: (e0 + e, 1, 0)),
      ],
      out_specs=pl.BlockSpec((1, 1, K, J), lambda e, b: (b, e0 + e, 0, 0)),
      out_shape=jax.ShapeDtypeStruct((B, E, K, J), jnp.float32),
      compiler_params=pltpu.CompilerParams(
          dimension_semantics=("arbitrary", "arbitrary"),
      ),
      **kwargs,
  )(y_prev, xg, W, W)


_EPC = 2


def kernel(x, Ind, W):
  B, T, I = x.shape
  E, K = Ind.shape[1], Ind.shape[2]
  xb = x.astype(jnp.bfloat16)
  lo = lax.bitcast_convert_type(xb[:, :, : I // 2], jnp.uint16).astype(jnp.uint32)
  hi = lax.bitcast_convert_type(xb[:, :, I // 2 :], jnp.uint16).astype(jnp.uint32)
  table = lax.bitcast_convert_type(lo | (hi << 16), jnp.int32).reshape(
      B * T, I // 2)
  flat_idx = (
      jnp.arange(B, dtype=jnp.int32)[None, :, None] * T
      + jnp.transpose(Ind, (1, 0, 2))
  ).reshape(E * B * K)

  nch = E // _EPC
  rows_per_chunk = _EPC * B * K
  xgs = [
      _sc_gather(table, flat_idx, c * rows_per_chunk, rows_per_chunk)
      for c in range(nch)
  ]
  y = None
  for c in range(nch):
    xg = xgs[c].reshape(_EPC, B, K, I // 2)
    if y is None:
      dummy = jnp.zeros((8, 128), jnp.float32)
      y = _tc_matmul_chunk(dummy, xg, W, 0, _EPC, alias=False)
    else:
      y = _tc_matmul_chunk(y, xg, W, c * _EPC, _EPC, alias=True)
  return y
